# baseline (speedup 1.0000x reference)
import functools

import jax
import jax.numpy as jnp
from jax.experimental import pallas as pl
from jax.experimental.pallas import tpu as pltpu

N, NFEAT, NHID, DHID1 = 4096, 128, 64, 32
BM_ENC = 256
BM_DEC = 256
NE = N // BM_ENC
ND = N // BM_DEC


def _body(adj_ref, x_ref, wgc_ref, bgc_ref, w1_ref, b1_ref, w2_ref,
          b2_ref, w3_ref, b3_ref, wdec_ref, out_ref, xw_scr, z_scr, zw_scr):
    i = pl.program_id(0)

    @pl.when(i == 0)
    def _():
        xw_scr[...] = jnp.dot(x_ref[...], wgc_ref[...],
                              preferred_element_type=jnp.float32)

    @pl.when(i < NE)
    def _():
        h = jnp.dot(adj_ref[...], xw_scr[...],
                    preferred_element_type=jnp.float32)
        h = jnp.maximum(h + bgc_ref[...], 0.0)
        h = jnp.maximum(jnp.dot(h, w1_ref[...],
                                preferred_element_type=jnp.float32)
                        + b1_ref[...], 0.0)
        h = jnp.maximum(jnp.dot(h, w2_ref[...],
                                preferred_element_type=jnp.float32)
                        + b2_ref[...], 0.0)
        z = (jnp.dot(h, w3_ref[...], preferred_element_type=jnp.float32)
             + b3_ref[...])
        z_scr[pl.ds(i * BM_ENC, BM_ENC), :] = z
        zw_scr[pl.ds(i * BM_ENC, BM_ENC), :] = jnp.dot(
            z, wdec_ref[...], preferred_element_type=jnp.float32)

    @pl.when(i >= NE)
    def _():
        j = i - NE
        out_ref[...] = jax.lax.dot_general(
            zw_scr[pl.ds(j * BM_DEC, BM_DEC), :], z_scr[...],
            (((1,), (1,)), ((), ())), preferred_element_type=jnp.float32)


@jax.jit
def kernel(x, adj_norm_pos, W_gc, b_gc, W1, b1, W2, b2, W3, b3, W_dec):
    full = lambda shape: pl.BlockSpec(shape, lambda i: (0,) * len(shape))

    logits = pl.pallas_call(
        _body,
        grid=(NE + ND,),
        in_specs=[
            pl.BlockSpec((BM_ENC, N), lambda i: (jax.lax.min(i, NE - 1), 0)),
            full((N, NFEAT)),
            full((NFEAT, NHID)),
            full((1, NHID)),
            full((NHID, DHID1)),
            full((1, DHID1)),
            full((DHID1, 2 * DHID1)),
            full((1, 2 * DHID1)),
            full((2 * DHID1, DHID1)),
            full((1, DHID1)),
            full((DHID1, DHID1)),
        ],
        out_specs=pl.BlockSpec((BM_DEC, N), lambda i: (jax.lax.max(i - NE, 0), 0)),
        out_shape=jax.ShapeDtypeStruct((N, N), jnp.float32),
        scratch_shapes=[
            pltpu.VMEM((N, NHID), jnp.float32),
            pltpu.VMEM((N, DHID1), jnp.float32),
            pltpu.VMEM((N, DHID1), jnp.float32),
        ],
        compiler_params=pltpu.CompilerParams(
            dimension_semantics=("arbitrary",)),
    )(adj_norm_pos, x, W_gc, b_gc.reshape(1, -1), W1, b1.reshape(1, -1),
      W2, b2.reshape(1, -1), W3, b3.reshape(1, -1), W_dec)
    return logits


# speedup vs baseline: 1.1199x; 1.1199x over previous
import functools

import jax
import jax.numpy as jnp
from jax.experimental import pallas as pl
from jax.experimental.pallas import tpu as pltpu

N, NFEAT, NHID, DHID1 = 4096, 128, 64, 32
BM_ENC = 1024
BM_DEC = 512
NE = N // BM_ENC
ND = N // BM_DEC


def _body(adj_ref, x_ref, wgc_ref, bgc_ref, w1_ref, b1_ref, w2_ref,
          b2_ref, w3_ref, b3_ref, wdec_ref, out_ref, xw_scr, z_scr, zw_scr):
    i = pl.program_id(0)

    @pl.when(i == 0)
    def _():
        xw_scr[...] = jnp.dot(x_ref[...], wgc_ref[...],
                              preferred_element_type=jnp.float32)

    @pl.when(i < NE)
    def _():
        h = jnp.dot(adj_ref[...], xw_scr[...],
                    preferred_element_type=jnp.float32)
        h = jnp.maximum(h + bgc_ref[...], 0.0)
        h = jnp.maximum(jnp.dot(h, w1_ref[...],
                                preferred_element_type=jnp.float32)
                        + b1_ref[...], 0.0)
        h = jnp.maximum(jnp.dot(h, w2_ref[...],
                                preferred_element_type=jnp.float32)
                        + b2_ref[...], 0.0)
        z = (jnp.dot(h, w3_ref[...], preferred_element_type=jnp.float32)
             + b3_ref[...])
        z_scr[pl.ds(i * BM_ENC, BM_ENC), :] = z
        zw_scr[pl.ds(i * BM_ENC, BM_ENC), :] = jnp.dot(
            z, wdec_ref[...], preferred_element_type=jnp.float32)

    @pl.when(i >= NE)
    def _():
        j = i - NE
        out_ref[...] = jax.lax.dot_general(
            zw_scr[pl.ds(j * BM_DEC, BM_DEC), :], z_scr[...],
            (((1,), (1,)), ((), ())), preferred_element_type=jnp.float32)


@jax.jit
def kernel(x, adj_norm_pos, W_gc, b_gc, W1, b1, W2, b2, W3, b3, W_dec):
    full = lambda shape: pl.BlockSpec(shape, lambda i: (0,) * len(shape))

    logits = pl.pallas_call(
        _body,
        grid=(NE + ND,),
        in_specs=[
            pl.BlockSpec((BM_ENC, N), lambda i: (jax.lax.min(i, NE - 1), 0)),
            full((N, NFEAT)),
            full((NFEAT, NHID)),
            full((1, NHID)),
            full((NHID, DHID1)),
            full((1, DHID1)),
            full((DHID1, 2 * DHID1)),
            full((1, 2 * DHID1)),
            full((2 * DHID1, DHID1)),
            full((1, DHID1)),
            full((DHID1, DHID1)),
        ],
        out_specs=pl.BlockSpec((BM_DEC, N), lambda i: (jax.lax.max(i - NE, 0), 0)),
        out_shape=jax.ShapeDtypeStruct((N, N), jnp.float32),
        scratch_shapes=[
            pltpu.VMEM((N, NHID), jnp.float32),
            pltpu.VMEM((N, DHID1), jnp.float32),
            pltpu.VMEM((N, DHID1), jnp.float32),
        ],
        compiler_params=pltpu.CompilerParams(
            dimension_semantics=("arbitrary",)),
    )(adj_norm_pos, x, W_gc, b_gc.reshape(1, -1), W1, b1.reshape(1, -1),
      W2, b2.reshape(1, -1), W3, b3.reshape(1, -1), W_dec)
    return logits
